# baseline (speedup 1.0000x reference)
import functools

import jax
import jax.numpy as jnp
from jax.experimental import pallas as pl
from jax.experimental.pallas import tpu as pltpu

_ALPHA = 0.2

_PARAMS = pltpu.CompilerParams(dimension_semantics=("arbitrary",))


def _leaky(x):
    return jnp.where(x > 0, x, _ALPHA * x)


def _relu(x):
    return jnp.maximum(x, 0.0)


def _pick_bm(m, candidates):
    for c in candidates:
        if m % c == 0:
            return c
    return m


def _bf(x):
    return x.astype(jnp.bfloat16)


def _lin_kernel(x_ref, w_ref, out_ref):
    out_ref[...] = _bf(jnp.dot(x_ref[...], w_ref[...],
                               preferred_element_type=jnp.float32))


def _linear_bf16(x, w, bm):
    m, d = x.shape
    n = w.shape[1]
    return pl.pallas_call(
        _lin_kernel,
        grid=(m // bm,),
        in_specs=[
            pl.BlockSpec((bm, d), lambda i: (i, 0)),
            pl.BlockSpec((d, n), lambda i: (0, 0)),
        ],
        out_specs=pl.BlockSpec((bm, n), lambda i: (i, 0)),
        out_shape=jax.ShapeDtypeStruct((m, n), jnp.bfloat16),
        compiler_params=_PARAMS,
    )(x, w)


def _pass1_kernel(adj_ref, sup_ref, b_ref, x_ref, wh_ref, bh_ref,
                  wpa_ref, wpb_ref, h0_ref, pair_ref):
    a16 = _bf(adj_ref[...])
    acc = jnp.dot(a16, sup_ref[...], preferred_element_type=jnp.float32)
    n = _leaky(acc + b_ref[...])
    d = n.shape[1]
    wh = wh_ref[...]
    h0 = _relu(jnp.dot(x_ref[...], wh[:d, :],
                       preferred_element_type=jnp.float32)
               + jnp.dot(n, wh[d:, :], preferred_element_type=jnp.float32)
               + bh_ref[...])
    h0_ref[...] = h0
    pa = jnp.dot(h0, wpa_ref[...], preferred_element_type=jnp.float32)
    pb = jnp.dot(n, wpb_ref[...], preferred_element_type=jnp.float32)
    pair_ref[...] = _bf(jnp.concatenate([pa, pb], axis=1))


def _pass1(adj, sup16, b, x, wh, bh, wpa, wpb, bm):
    m, k = adj.shape
    d = sup16.shape[1]
    return pl.pallas_call(
        _pass1_kernel,
        grid=(m // bm,),
        in_specs=[
            pl.BlockSpec((bm, k), lambda i: (i, 0)),
            pl.BlockSpec((k, d), lambda i: (0, 0)),
            pl.BlockSpec((1, d), lambda i: (0, 0)),
            pl.BlockSpec((bm, d), lambda i: (i, 0)),
            pl.BlockSpec((2 * d, d), lambda i: (0, 0)),
            pl.BlockSpec((1, d), lambda i: (0, 0)),
            pl.BlockSpec((d, d), lambda i: (0, 0)),
            pl.BlockSpec((d, d), lambda i: (0, 0)),
        ],
        out_specs=[
            pl.BlockSpec((bm, d), lambda i: (i, 0)),
            pl.BlockSpec((bm, 2 * d), lambda i: (i, 0)),
        ],
        out_shape=[
            jax.ShapeDtypeStruct((m, d), jnp.float32),
            jax.ShapeDtypeStruct((m, 2 * d), jnp.bfloat16),
        ],
        compiler_params=_PARAMS,
    )(adj, sup16, b.reshape(1, d), x, wh, bh.reshape(1, d), wpa, wpb)


def _merged_kernel(adj_ref, sup_ref, b0_ref, x_ref, wh_ref, bh_ref,
                   wpa_ref, wpb_ref, b1_ref, wl_ref, bl_ref,
                   h0_ref, pair_ref, learn_ref, hmean_ref, *, inv_m, d):
    a16 = _bf(adj_ref[...])
    acc = jnp.dot(a16, sup_ref[...], preferred_element_type=jnp.float32)
    n = _leaky(acc[:, :d] + b0_ref[...])
    wh = wh_ref[...]
    h0 = _relu(jnp.dot(x_ref[...], wh[:d, :],
                       preferred_element_type=jnp.float32)
               + jnp.dot(n, wh[d:, :], preferred_element_type=jnp.float32)
               + bh_ref[...])
    h0_ref[...] = h0
    pa = jnp.dot(h0, wpa_ref[...], preferred_element_type=jnp.float32)
    pb = jnp.dot(n, wpb_ref[...], preferred_element_type=jnp.float32)
    pair_ref[...] = _bf(jnp.concatenate([pa, pb], axis=1))

    un = _leaky(acc[:, d:] + b1_ref[...])
    n1 = un[:, :d]
    n2 = un[:, d:]
    wl = wl_ref[...]
    learn = _relu(
        jnp.dot(h0, wl[:d, :], preferred_element_type=jnp.float32)
        + jnp.dot(n2, wl[d:2 * d, :], preferred_element_type=jnp.float32)
        + jnp.dot(n1, wl[2 * d:, :], preferred_element_type=jnp.float32)
        + bl_ref[...])
    learn_ref[...] = learn
    part = jnp.concatenate([
        jnp.sum(h0, axis=0, keepdims=True) * inv_m,
        jnp.sum(learn, axis=0, keepdims=True) * inv_m,
    ], axis=0)

    @pl.when(pl.program_id(0) == 0)
    def _init():
        hmean_ref[...] = part

    @pl.when(pl.program_id(0) != 0)
    def _acc():
        hmean_ref[...] += part


def _merged(adj, sup_all16, b0, x, wh, bh, wpa, wpb, b1, wl, bl, bm):
    m, k = adj.shape
    d = b0.shape[0]
    return pl.pallas_call(
        functools.partial(_merged_kernel, inv_m=1.0 / m, d=d),
        grid=(m // bm,),
        in_specs=[
            pl.BlockSpec((bm, k), lambda i: (i, 0)),
            pl.BlockSpec((k, 3 * d), lambda i: (0, 0)),
            pl.BlockSpec((1, d), lambda i: (0, 0)),
            pl.BlockSpec((bm, d), lambda i: (i, 0)),
            pl.BlockSpec((2 * d, d), lambda i: (0, 0)),
            pl.BlockSpec((1, d), lambda i: (0, 0)),
            pl.BlockSpec((d, d), lambda i: (0, 0)),
            pl.BlockSpec((d, d), lambda i: (0, 0)),
            pl.BlockSpec((1, 2 * d), lambda i: (0, 0)),
            pl.BlockSpec((3 * d, d), lambda i: (0, 0)),
            pl.BlockSpec((1, d), lambda i: (0, 0)),
        ],
        out_specs=[
            pl.BlockSpec((bm, d), lambda i: (i, 0)),
            pl.BlockSpec((bm, 2 * d), lambda i: (i, 0)),
            pl.BlockSpec((bm, d), lambda i: (i, 0)),
            pl.BlockSpec((2, d), lambda i: (0, 0)),
        ],
        out_shape=[
            jax.ShapeDtypeStruct((m, d), jnp.float32),
            jax.ShapeDtypeStruct((m, 2 * d), jnp.bfloat16),
            jax.ShapeDtypeStruct((m, d), jnp.float32),
            jax.ShapeDtypeStruct((2, d), jnp.float32),
        ],
        compiler_params=_PARAMS,
    )(adj, sup_all16, b0.reshape(1, d), x, wh, bh.reshape(1, d), wpa, wpb,
      b1.reshape(1, 2 * d), wl, bl.reshape(1, d))


def _pass2_kernel(adj_ref, sup_ref, b_ref, h0_ref, w_ref, bl_ref,
                  learn_ref, hmean_ref, *, inv_m):
    a16 = _bf(adj_ref[...])
    acc = jnp.dot(a16, sup_ref[...], preferred_element_type=jnp.float32)
    un = _leaky(acc + b_ref[...])
    d = h0_ref.shape[1]
    n1 = un[:, :d]
    n2 = un[:, d:]
    w = w_ref[...]
    h0 = h0_ref[...]
    learn = _relu(
        jnp.dot(h0, w[:d, :], preferred_element_type=jnp.float32)
        + jnp.dot(n2, w[d:2 * d, :], preferred_element_type=jnp.float32)
        + jnp.dot(n1, w[2 * d:, :], preferred_element_type=jnp.float32)
        + bl_ref[...])
    learn_ref[...] = learn
    part = jnp.concatenate([
        jnp.sum(h0, axis=0, keepdims=True) * inv_m,
        jnp.sum(learn, axis=0, keepdims=True) * inv_m,
    ], axis=0)

    @pl.when(pl.program_id(0) == 0)
    def _init():
        hmean_ref[...] = part

    @pl.when(pl.program_id(0) != 0)
    def _acc():
        hmean_ref[...] += part


def _pass2(adj, sup16, bcat, h0, w, bl, bm):
    m, k = adj.shape
    d = h0.shape[1]
    return pl.pallas_call(
        functools.partial(_pass2_kernel, inv_m=1.0 / m),
        grid=(m // bm,),
        in_specs=[
            pl.BlockSpec((bm, k), lambda i: (i, 0)),
            pl.BlockSpec((k, 2 * d), lambda i: (0, 0)),
            pl.BlockSpec((1, 2 * d), lambda i: (0, 0)),
            pl.BlockSpec((bm, d), lambda i: (i, 0)),
            pl.BlockSpec((3 * d, d), lambda i: (0, 0)),
            pl.BlockSpec((1, d), lambda i: (0, 0)),
        ],
        out_specs=[
            pl.BlockSpec((bm, d), lambda i: (i, 0)),
            pl.BlockSpec((2, d), lambda i: (0, 0)),
        ],
        out_shape=[
            jax.ShapeDtypeStruct((m, d), jnp.float32),
            jax.ShapeDtypeStruct((2, d), jnp.float32),
        ],
        compiler_params=_PARAMS,
    )(adj, sup16, bcat.reshape(1, 2 * d), h0, w, bl.reshape(1, d))


def _tail_kernel(h_ref, w_ref, b_ref, w2_ref, h0_ref, l_ref,
                 fin_ref, a_ref):
    z = _relu(jnp.dot(h_ref[...], w_ref[...],
                      preferred_element_type=jnp.float32) + b_ref[...])
    logits = jnp.sum(z * w2_ref[...], axis=1, keepdims=True)
    mx = jnp.max(logits, axis=0, keepdims=True)
    e = jnp.exp(logits - mx)
    alpha = e / jnp.sum(e, axis=0, keepdims=True)
    fin_ref[...] = 0.5 * (alpha[0:1, :] * h0_ref[...]
                          + alpha[1:2, :] * l_ref[...])

    @pl.when(pl.program_id(0) == 0)
    def _emit_alpha():
        a_ref[...] = jnp.broadcast_to(alpha, a_ref.shape)


def _tail(h, w, b, w2, h0, learn, bm):
    m, d = h0.shape
    full = lambda i: (0, 0)
    blk = lambda i: (i, 0)
    return pl.pallas_call(
        _tail_kernel,
        grid=(m // bm,),
        in_specs=[
            pl.BlockSpec((2, d), full),
            pl.BlockSpec((d, d), full),
            pl.BlockSpec((1, d), full),
            pl.BlockSpec((1, d), full),
            pl.BlockSpec((bm, d), blk),
            pl.BlockSpec((bm, d), blk),
        ],
        out_specs=[
            pl.BlockSpec((bm, d), blk),
            pl.BlockSpec((2, d), full),
        ],
        out_shape=[
            jax.ShapeDtypeStruct((m, d), jnp.float32),
            jax.ShapeDtypeStruct((2, d), jnp.float32),
        ],
        compiler_params=_PARAMS,
    )(h, w, b.reshape(1, d), w2.reshape(1, d), h0, learn)


def kernel(ufea, vfea, UV_adj, VU_adj, adj, params):
    p = params
    n_u = ufea.shape[0]
    n_i = vfea.shape[0]

    bm_u = _pick_bm(n_u, [400, 200, 80, 40, 16, 8])
    bm_i = _pick_bm(n_i, [400, 200, 80, 40, 16, 8])
    bm_lin_u = _pick_bm(n_u, [2000, 1000, 400, 80, 16, 8])
    bm_lin_i = _pick_bm(n_i, [2000, 1000, 400, 80, 16, 8])

    sup_u16 = _linear_bf16(vfea, p['W_gc1_0'], bm_lin_i)
    sup_i16 = _linear_bf16(ufea, p['W_gc2_0'], bm_lin_u)

    b_uv = jnp.concatenate([p['b_gc3_1'], p['b_gc1_1']])
    b_vu = jnp.concatenate([p['b_gc4_1'], p['b_gc2_1']])

    Item_h0, sup_uv16 = _pass1(VU_adj, sup_i16, p['b_gc2_0'], vfea,
                               p['W_iu0'], p['b_iu0'],
                               p['W_gc3_1'], p['W_gc1_1'], bm_i)

    sup_all16 = jnp.concatenate([sup_u16, sup_uv16], axis=1)
    User_h0, sup_vu16, learn_user, Hu = _merged(
        UV_adj, sup_all16, p['b_gc1_0'], ufea, p['W_uu0'], p['b_uu0'],
        p['W_gc4_1'], p['W_gc2_1'], b_uv,
        p['W_uu1'], p['b_uu1'], bm_u)

    learn_item, Hv = _pass2(VU_adj, sup_vu16, b_vu, Item_h0,
                            p['W_iu1'], p['b_iu1'], bm_i)

    h_u_final, alpha_u_bc = _tail(Hu, p['W_mlp_ul'], p['b_mlp_ul'],
                                  p['W_mlp_ul1'], User_h0, learn_user,
                                  bm_lin_u)
    h_v_final, alpha_v_bc = _tail(Hv, p['W_mlp_vl'], p['b_mlp_vl'],
                                  p['W_mlp_vl1'], Item_h0, learn_item,
                                  bm_lin_i)

    alpha_ul = alpha_u_bc[:, :1]
    alpha_vl = alpha_v_bc[:, :1]

    return (learn_user, learn_item, h_u_final, h_v_final,
            alpha_ul, alpha_vl, Hu, Hv)


# speedup vs baseline: 1.0261x; 1.0261x over previous
import functools

import jax
import jax.numpy as jnp
from jax.experimental import pallas as pl
from jax.experimental.pallas import tpu as pltpu

_ALPHA = 0.2

_PARAMS = pltpu.CompilerParams(dimension_semantics=("arbitrary",))


def _leaky(x):
    return jnp.where(x > 0, x, _ALPHA * x)


def _relu(x):
    return jnp.maximum(x, 0.0)


def _pick_bm(m, candidates):
    for c in candidates:
        if m % c == 0:
            return c
    return m


def _bf(x):
    return x.astype(jnp.bfloat16)


def _lin_kernel(x_ref, w_ref, out_ref):
    out_ref[...] = _bf(jnp.dot(x_ref[...], w_ref[...],
                               preferred_element_type=jnp.float32))


def _linear_bf16(x, w, bm):
    m, d = x.shape
    n = w.shape[1]
    return pl.pallas_call(
        _lin_kernel,
        grid=(m // bm,),
        in_specs=[
            pl.BlockSpec((bm, d), lambda i: (i, 0)),
            pl.BlockSpec((d, n), lambda i: (0, 0)),
        ],
        out_specs=pl.BlockSpec((bm, n), lambda i: (i, 0)),
        out_shape=jax.ShapeDtypeStruct((m, n), jnp.bfloat16),
        compiler_params=_PARAMS,
    )(x, w)


def _pass1_kernel(adj_ref, sup_ref, b_ref, x_ref, wh_ref, bh_ref,
                  wpa_ref, wpb_ref, wnext_ref, h0_ref, sup_all_ref):
    a16 = _bf(adj_ref[...])
    acc = jnp.dot(a16, sup_ref[...], preferred_element_type=jnp.float32)
    n = _leaky(acc + b_ref[...])
    d = n.shape[1]
    x = x_ref[...]
    wh = wh_ref[...]
    h0 = _relu(jnp.dot(x, wh[:d, :], preferred_element_type=jnp.float32)
               + jnp.dot(n, wh[d:, :], preferred_element_type=jnp.float32)
               + bh_ref[...])
    h0_ref[...] = h0
    s0 = jnp.dot(x, wnext_ref[...], preferred_element_type=jnp.float32)
    pa = jnp.dot(h0, wpa_ref[...], preferred_element_type=jnp.float32)
    pb = jnp.dot(n, wpb_ref[...], preferred_element_type=jnp.float32)
    sup_all_ref[...] = _bf(jnp.concatenate([s0, pa, pb], axis=1))


def _pass1(adj, sup16, b, x, wh, bh, wpa, wpb, wnext, bm):
    m, k = adj.shape
    d = sup16.shape[1]
    return pl.pallas_call(
        _pass1_kernel,
        grid=(m // bm,),
        in_specs=[
            pl.BlockSpec((bm, k), lambda i: (i, 0)),
            pl.BlockSpec((k, d), lambda i: (0, 0)),
            pl.BlockSpec((1, d), lambda i: (0, 0)),
            pl.BlockSpec((bm, d), lambda i: (i, 0)),
            pl.BlockSpec((2 * d, d), lambda i: (0, 0)),
            pl.BlockSpec((1, d), lambda i: (0, 0)),
            pl.BlockSpec((d, d), lambda i: (0, 0)),
            pl.BlockSpec((d, d), lambda i: (0, 0)),
            pl.BlockSpec((d, d), lambda i: (0, 0)),
        ],
        out_specs=[
            pl.BlockSpec((bm, d), lambda i: (i, 0)),
            pl.BlockSpec((bm, 3 * d), lambda i: (i, 0)),
        ],
        out_shape=[
            jax.ShapeDtypeStruct((m, d), jnp.float32),
            jax.ShapeDtypeStruct((m, 3 * d), jnp.bfloat16),
        ],
        compiler_params=_PARAMS,
    )(adj, sup16, b.reshape(1, d), x, wh, bh.reshape(1, d), wpa, wpb, wnext)


def _merged_kernel(adj_ref, sup_ref, b0_ref, x_ref, wh_ref, bh_ref,
                   wpa_ref, wpb_ref, b1_ref, wl_ref, bl_ref,
                   h0_ref, pair_ref, learn_ref, hmean_ref, *, inv_m, d):
    a16 = _bf(adj_ref[...])
    acc = jnp.dot(a16, sup_ref[...], preferred_element_type=jnp.float32)
    n = _leaky(acc[:, :d] + b0_ref[...])
    wh = wh_ref[...]
    h0 = _relu(jnp.dot(x_ref[...], wh[:d, :],
                       preferred_element_type=jnp.float32)
               + jnp.dot(n, wh[d:, :], preferred_element_type=jnp.float32)
               + bh_ref[...])
    h0_ref[...] = h0
    pa = jnp.dot(h0, wpa_ref[...], preferred_element_type=jnp.float32)
    pb = jnp.dot(n, wpb_ref[...], preferred_element_type=jnp.float32)
    pair_ref[...] = _bf(jnp.concatenate([pa, pb], axis=1))

    un = _leaky(acc[:, d:] + b1_ref[...])
    n1 = un[:, :d]
    n2 = un[:, d:]
    wl = wl_ref[...]
    learn = _relu(
        jnp.dot(h0, wl[:d, :], preferred_element_type=jnp.float32)
        + jnp.dot(n2, wl[d:2 * d, :], preferred_element_type=jnp.float32)
        + jnp.dot(n1, wl[2 * d:, :], preferred_element_type=jnp.float32)
        + bl_ref[...])
    learn_ref[...] = learn
    part = jnp.concatenate([
        jnp.sum(h0, axis=0, keepdims=True) * inv_m,
        jnp.sum(learn, axis=0, keepdims=True) * inv_m,
    ], axis=0)

    @pl.when(pl.program_id(0) == 0)
    def _init():
        hmean_ref[...] = part

    @pl.when(pl.program_id(0) != 0)
    def _acc():
        hmean_ref[...] += part


def _merged(adj, sup_all16, b0, x, wh, bh, wpa, wpb, b1, wl, bl, bm):
    m, k = adj.shape
    d = b0.shape[0]
    return pl.pallas_call(
        functools.partial(_merged_kernel, inv_m=1.0 / m, d=d),
        grid=(m // bm,),
        in_specs=[
            pl.BlockSpec((bm, k), lambda i: (i, 0)),
            pl.BlockSpec((k, 3 * d), lambda i: (0, 0)),
            pl.BlockSpec((1, d), lambda i: (0, 0)),
            pl.BlockSpec((bm, d), lambda i: (i, 0)),
            pl.BlockSpec((2 * d, d), lambda i: (0, 0)),
            pl.BlockSpec((1, d), lambda i: (0, 0)),
            pl.BlockSpec((d, d), lambda i: (0, 0)),
            pl.BlockSpec((d, d), lambda i: (0, 0)),
            pl.BlockSpec((1, 2 * d), lambda i: (0, 0)),
            pl.BlockSpec((3 * d, d), lambda i: (0, 0)),
            pl.BlockSpec((1, d), lambda i: (0, 0)),
        ],
        out_specs=[
            pl.BlockSpec((bm, d), lambda i: (i, 0)),
            pl.BlockSpec((bm, 2 * d), lambda i: (i, 0)),
            pl.BlockSpec((bm, d), lambda i: (i, 0)),
            pl.BlockSpec((2, d), lambda i: (0, 0)),
        ],
        out_shape=[
            jax.ShapeDtypeStruct((m, d), jnp.float32),
            jax.ShapeDtypeStruct((m, 2 * d), jnp.bfloat16),
            jax.ShapeDtypeStruct((m, d), jnp.float32),
            jax.ShapeDtypeStruct((2, d), jnp.float32),
        ],
        compiler_params=_PARAMS,
    )(adj, sup_all16, b0.reshape(1, d), x, wh, bh.reshape(1, d), wpa, wpb,
      b1.reshape(1, 2 * d), wl, bl.reshape(1, d))


def _pass2_kernel(adj_ref, sup_ref, b_ref, h0_ref, w_ref, bl_ref,
                  learn_ref, hmean_ref, *, inv_m):
    a16 = _bf(adj_ref[...])
    acc = jnp.dot(a16, sup_ref[...], preferred_element_type=jnp.float32)
    un = _leaky(acc + b_ref[...])
    d = h0_ref.shape[1]
    n1 = un[:, :d]
    n2 = un[:, d:]
    w = w_ref[...]
    h0 = h0_ref[...]
    learn = _relu(
        jnp.dot(h0, w[:d, :], preferred_element_type=jnp.float32)
        + jnp.dot(n2, w[d:2 * d, :], preferred_element_type=jnp.float32)
        + jnp.dot(n1, w[2 * d:, :], preferred_element_type=jnp.float32)
        + bl_ref[...])
    learn_ref[...] = learn
    part = jnp.concatenate([
        jnp.sum(h0, axis=0, keepdims=True) * inv_m,
        jnp.sum(learn, axis=0, keepdims=True) * inv_m,
    ], axis=0)

    @pl.when(pl.program_id(0) == 0)
    def _init():
        hmean_ref[...] = part

    @pl.when(pl.program_id(0) != 0)
    def _acc():
        hmean_ref[...] += part


def _pass2(adj, sup16, bcat, h0, w, bl, bm):
    m, k = adj.shape
    d = h0.shape[1]
    return pl.pallas_call(
        functools.partial(_pass2_kernel, inv_m=1.0 / m),
        grid=(m // bm,),
        in_specs=[
            pl.BlockSpec((bm, k), lambda i: (i, 0)),
            pl.BlockSpec((k, 2 * d), lambda i: (0, 0)),
            pl.BlockSpec((1, 2 * d), lambda i: (0, 0)),
            pl.BlockSpec((bm, d), lambda i: (i, 0)),
            pl.BlockSpec((3 * d, d), lambda i: (0, 0)),
            pl.BlockSpec((1, d), lambda i: (0, 0)),
        ],
        out_specs=[
            pl.BlockSpec((bm, d), lambda i: (i, 0)),
            pl.BlockSpec((2, d), lambda i: (0, 0)),
        ],
        out_shape=[
            jax.ShapeDtypeStruct((m, d), jnp.float32),
            jax.ShapeDtypeStruct((2, d), jnp.float32),
        ],
        compiler_params=_PARAMS,
    )(adj, sup16, bcat.reshape(1, 2 * d), h0, w, bl.reshape(1, d))


def _tail_kernel(h_ref, w_ref, b_ref, w2_ref, h0_ref, l_ref,
                 fin_ref, a_ref):
    z = _relu(jnp.dot(h_ref[...], w_ref[...],
                      preferred_element_type=jnp.float32) + b_ref[...])
    logits = jnp.sum(z * w2_ref[...], axis=1, keepdims=True)
    mx = jnp.max(logits, axis=0, keepdims=True)
    e = jnp.exp(logits - mx)
    alpha = e / jnp.sum(e, axis=0, keepdims=True)
    fin_ref[...] = 0.5 * (alpha[0:1, :] * h0_ref[...]
                          + alpha[1:2, :] * l_ref[...])

    @pl.when(pl.program_id(0) == 0)
    def _emit_alpha():
        a_ref[...] = jnp.broadcast_to(alpha, a_ref.shape)


def _tail(h, w, b, w2, h0, learn, bm):
    m, d = h0.shape
    full = lambda i: (0, 0)
    blk = lambda i: (i, 0)
    return pl.pallas_call(
        _tail_kernel,
        grid=(m // bm,),
        in_specs=[
            pl.BlockSpec((2, d), full),
            pl.BlockSpec((d, d), full),
            pl.BlockSpec((1, d), full),
            pl.BlockSpec((1, d), full),
            pl.BlockSpec((bm, d), blk),
            pl.BlockSpec((bm, d), blk),
        ],
        out_specs=[
            pl.BlockSpec((bm, d), blk),
            pl.BlockSpec((2, d), full),
        ],
        out_shape=[
            jax.ShapeDtypeStruct((m, d), jnp.float32),
            jax.ShapeDtypeStruct((2, d), jnp.float32),
        ],
        compiler_params=_PARAMS,
    )(h, w, b.reshape(1, d), w2.reshape(1, d), h0, learn)


def kernel(ufea, vfea, UV_adj, VU_adj, adj, params):
    p = params
    n_u = ufea.shape[0]
    n_i = vfea.shape[0]

    bm_u = _pick_bm(n_u, [400, 200, 80, 40, 16, 8])
    bm_i = _pick_bm(n_i, [400, 200, 80, 40, 16, 8])
    bm_lin_u = _pick_bm(n_u, [2000, 1000, 400, 80, 16, 8])
    bm_lin_i = _pick_bm(n_i, [2000, 1000, 400, 80, 16, 8])

    sup_i16 = _linear_bf16(ufea, p['W_gc2_0'], bm_lin_u)

    b_uv = jnp.concatenate([p['b_gc3_1'], p['b_gc1_1']])
    b_vu = jnp.concatenate([p['b_gc4_1'], p['b_gc2_1']])

    Item_h0, sup_all16 = _pass1(VU_adj, sup_i16, p['b_gc2_0'], vfea,
                                p['W_iu0'], p['b_iu0'],
                                p['W_gc3_1'], p['W_gc1_1'],
                                p['W_gc1_0'], bm_i)

    User_h0, sup_vu16, learn_user, Hu = _merged(
        UV_adj, sup_all16, p['b_gc1_0'], ufea, p['W_uu0'], p['b_uu0'],
        p['W_gc4_1'], p['W_gc2_1'], b_uv,
        p['W_uu1'], p['b_uu1'], bm_u)

    learn_item, Hv = _pass2(VU_adj, sup_vu16, b_vu, Item_h0,
                            p['W_iu1'], p['b_iu1'], bm_i)

    h_u_final, alpha_u_bc = _tail(Hu, p['W_mlp_ul'], p['b_mlp_ul'],
                                  p['W_mlp_ul1'], User_h0, learn_user,
                                  bm_lin_u)
    h_v_final, alpha_v_bc = _tail(Hv, p['W_mlp_vl'], p['b_mlp_vl'],
                                  p['W_mlp_vl1'], Item_h0, learn_item,
                                  bm_lin_i)

    alpha_ul = alpha_u_bc[:, :1]
    alpha_vl = alpha_v_bc[:, :1]

    return (learn_user, learn_item, h_u_final, h_v_final,
            alpha_ul, alpha_vl, Hu, Hv)
